# baseline (speedup 1.0000x reference)
import functools

import jax
import jax.numpy as jnp
from jax import lax
from jax.experimental import pallas as pl
from jax.experimental.pallas import tpu as pltpu
from jax.experimental.pallas import tpu_sc as plsc

BATCH = 16384
EMBED_DIM = 64
LANES = 16
CHUNK = 128

_GDN = lax.GatherDimensionNumbers(
    offset_dims=(), collapsed_slice_dims=(0,), start_index_map=(0,))


def _shuffle(v, idx):
    return lax.gather(v, idx[:, None], _GDN, slice_sizes=(1,),
                      mode=lax.GatherScatterMode.PROMISE_IN_BOUNDS)


def _normalize_rows(rows_v, n_rows):
    def body(r, carry):
        lane = lax.iota(jnp.int32, LANES)
        v0 = rows_v[r, pl.ds(0, LANES)]
        v1 = rows_v[r, pl.ds(LANES, LANES)]
        v2 = rows_v[r, pl.ds(2 * LANES, LANES)]
        v3 = rows_v[r, pl.ds(3 * LANES, LANES)]
        p = v0 * v0 + v1 * v1 + v2 * v2 + v3 * v3
        p = p + _shuffle(p, lane ^ 8)
        p = p + _shuffle(p, lane ^ 4)
        p = p + _shuffle(p, lane ^ 2)
        ssv = p + _shuffle(p, lane ^ 1)
        y = plsc.bitcast(0x5F3759DF - (plsc.bitcast(ssv, jnp.int32) >> 1),
                         jnp.float32)
        h = ssv * 0.5
        y = y * (1.5 - h * y * y)
        y = y * (1.5 - h * y * y)
        y = y * (1.5 - h * y * y)
        y = jnp.minimum(y, 1e12)
        rows_v[r, pl.ds(0, LANES)] = v0 * y
        rows_v[r, pl.ds(LANES, LANES)] = v1 * y
        rows_v[r, pl.ds(2 * LANES, LANES)] = v2 * y
        rows_v[r, pl.ds(3 * LANES, LANES)] = v3 * y
        return carry

    lax.fori_loop(0, n_rows, body, 0)


def _two_tower_sc(c_ids, p_ids, c_tab, p_tab, u_out, i_out,
                  cidx_v, pidx_v, crows_v, prows_v, csem, psem):
    info = plsc.get_sparse_core_info()
    nc = info.num_cores
    wid = lax.axis_index("s") * nc + lax.axis_index("c")
    b_per_w = BATCH // (nc * info.num_subcores)
    n_chunks = b_per_w // CHUNK
    base = wid * b_per_w

    for c in range(n_chunks):
        pltpu.sync_copy(c_ids.at[pl.ds(base + c * CHUNK, CHUNK)],
                        cidx_v.at[c])
        pltpu.sync_copy(p_ids.at[pl.ds(base + c * CHUNK, CHUNK)],
                        pidx_v.at[c])

    ccopies = [
        pltpu.async_copy(c_tab.at[cidx_v.at[c]],
                         crows_v.at[pl.ds(c * CHUNK, CHUNK)], csem)
        for c in range(n_chunks)
    ]
    pcopies = [
        pltpu.async_copy(p_tab.at[pidx_v.at[c]],
                         prows_v.at[pl.ds(c * CHUNK, CHUNK)], psem)
        for c in range(n_chunks)
    ]
    for cp in ccopies:
        cp.wait()
    _normalize_rows(crows_v, b_per_w)
    pltpu.sync_copy(crows_v, u_out.at[pl.ds(base, b_per_w)])
    for cp in pcopies:
        cp.wait()
    _normalize_rows(prows_v, b_per_w)
    pltpu.sync_copy(prows_v, i_out.at[pl.ds(base, b_per_w)])


def kernel(claimant_ids, provider_ids, claimant_table, provider_table):
    info = plsc.get_sparse_core_info()
    b_per_w = BATCH // (info.num_cores * info.num_subcores)
    n_chunks = b_per_w // CHUNK
    mesh = plsc.VectorSubcoreMesh(core_axis_name="c", subcore_axis_name="s")
    out_type = (
        jax.ShapeDtypeStruct((BATCH, EMBED_DIM), jnp.float32),
        jax.ShapeDtypeStruct((BATCH, EMBED_DIM), jnp.float32),
    )
    run = pl.kernel(
        _two_tower_sc,
        mesh=mesh,
        out_type=out_type,
        scratch_types=[
            pltpu.VMEM((n_chunks, CHUNK), jnp.int32),
            pltpu.VMEM((n_chunks, CHUNK), jnp.int32),
            pltpu.VMEM((b_per_w, EMBED_DIM), jnp.float32),
            pltpu.VMEM((b_per_w, EMBED_DIM), jnp.float32),
            pltpu.SemaphoreType.DMA,
            pltpu.SemaphoreType.DMA,
        ],
        compiler_params=pltpu.CompilerParams(
            needs_layout_passes=False, use_tc_tiling_on_sc=False),
    )
    return run(claimant_ids.astype(jnp.int32), provider_ids.astype(jnp.int32),
               claimant_table, provider_table)


# speedup vs baseline: 1.0206x; 1.0206x over previous
import functools

import jax
import jax.numpy as jnp
from jax import lax
from jax.experimental import pallas as pl
from jax.experimental.pallas import tpu as pltpu
from jax.experimental.pallas import tpu_sc as plsc

BATCH = 16384
EMBED_DIM = 64
LANES = 16
CHUNK = 128

_GDN = lax.GatherDimensionNumbers(
    offset_dims=(), collapsed_slice_dims=(0,), start_index_map=(0,))


def _shuffle(v, idx):
    return lax.gather(v, idx[:, None], _GDN, slice_sizes=(1,),
                      mode=lax.GatherScatterMode.PROMISE_IN_BOUNDS)


_ROWS_PER_STEP = 8


def _normalize_one_row(rows_v, r, lane):
    v0 = rows_v[r, pl.ds(0, LANES)]
    v1 = rows_v[r, pl.ds(LANES, LANES)]
    v2 = rows_v[r, pl.ds(2 * LANES, LANES)]
    v3 = rows_v[r, pl.ds(3 * LANES, LANES)]
    p = v0 * v0 + v1 * v1 + v2 * v2 + v3 * v3
    p = p + _shuffle(p, lane ^ 8)
    p = p + _shuffle(p, lane ^ 4)
    p = p + _shuffle(p, lane ^ 2)
    ssv = p + _shuffle(p, lane ^ 1)
    y = plsc.bitcast(0x5F3759DF - (plsc.bitcast(ssv, jnp.int32) >> 1),
                     jnp.float32)
    h = ssv * 0.5
    y = y * (1.5 - h * y * y)
    y = y * (1.5 - h * y * y)
    y = y * (1.5 - h * y * y)
    y = jnp.minimum(y, 1e12)
    rows_v[r, pl.ds(0, LANES)] = v0 * y
    rows_v[r, pl.ds(LANES, LANES)] = v1 * y
    rows_v[r, pl.ds(2 * LANES, LANES)] = v2 * y
    rows_v[r, pl.ds(3 * LANES, LANES)] = v3 * y


def _normalize_rows(rows_v, n_rows):
    lane = lax.iota(jnp.int32, LANES)

    @plsc.parallel_loop(0, n_rows, _ROWS_PER_STEP, unroll=2)
    def body(r):
        for j in range(_ROWS_PER_STEP):
            _normalize_one_row(rows_v, r + j, lane)


def _two_tower_sc(c_ids, p_ids, c_tab, p_tab, u_out, i_out,
                  cidx_v, pidx_v, crows_v, prows_v, csem, psem):
    info = plsc.get_sparse_core_info()
    nc = info.num_cores
    wid = lax.axis_index("s") * nc + lax.axis_index("c")
    b_per_w = BATCH // (nc * info.num_subcores)
    n_chunks = b_per_w // CHUNK
    base = wid * b_per_w

    for c in range(n_chunks):
        pltpu.sync_copy(c_ids.at[pl.ds(base + c * CHUNK, CHUNK)],
                        cidx_v.at[c])
        pltpu.sync_copy(p_ids.at[pl.ds(base + c * CHUNK, CHUNK)],
                        pidx_v.at[c])

    ccopies = [
        pltpu.async_copy(c_tab.at[cidx_v.at[c]],
                         crows_v.at[pl.ds(c * CHUNK, CHUNK)], csem)
        for c in range(n_chunks)
    ]
    pcopies = [
        pltpu.async_copy(p_tab.at[pidx_v.at[c]],
                         prows_v.at[pl.ds(c * CHUNK, CHUNK)], psem)
        for c in range(n_chunks)
    ]
    for cp in ccopies:
        cp.wait()
    _normalize_rows(crows_v, b_per_w)
    pltpu.sync_copy(crows_v, u_out.at[pl.ds(base, b_per_w)])
    for cp in pcopies:
        cp.wait()
    _normalize_rows(prows_v, b_per_w)
    pltpu.sync_copy(prows_v, i_out.at[pl.ds(base, b_per_w)])


def kernel(claimant_ids, provider_ids, claimant_table, provider_table):
    info = plsc.get_sparse_core_info()
    b_per_w = BATCH // (info.num_cores * info.num_subcores)
    n_chunks = b_per_w // CHUNK
    mesh = plsc.VectorSubcoreMesh(core_axis_name="c", subcore_axis_name="s")
    out_type = (
        jax.ShapeDtypeStruct((BATCH, EMBED_DIM), jnp.float32),
        jax.ShapeDtypeStruct((BATCH, EMBED_DIM), jnp.float32),
    )
    run = pl.kernel(
        _two_tower_sc,
        mesh=mesh,
        out_type=out_type,
        scratch_types=[
            pltpu.VMEM((n_chunks, CHUNK), jnp.int32),
            pltpu.VMEM((n_chunks, CHUNK), jnp.int32),
            pltpu.VMEM((b_per_w, EMBED_DIM), jnp.float32),
            pltpu.VMEM((b_per_w, EMBED_DIM), jnp.float32),
            pltpu.SemaphoreType.DMA,
            pltpu.SemaphoreType.DMA,
        ],
        compiler_params=pltpu.CompilerParams(
            needs_layout_passes=False, use_tc_tiling_on_sc=False),
    )
    return run(claimant_ids.astype(jnp.int32), provider_ids.astype(jnp.int32),
               claimant_table, provider_table)


# speedup vs baseline: 1.0781x; 1.0563x over previous
import functools

import jax
import jax.numpy as jnp
from jax import lax
from jax.experimental import pallas as pl
from jax.experimental.pallas import tpu as pltpu
from jax.experimental.pallas import tpu_sc as plsc

BATCH = 16384
EMBED_DIM = 64
PAD_DIM = 128
LANES = 16
CHUNK = 128

_GDN = lax.GatherDimensionNumbers(
    offset_dims=(), collapsed_slice_dims=(0,), start_index_map=(0,))


def _shuffle(v, idx):
    return lax.gather(v, idx[:, None], _GDN, slice_sizes=(1,),
                      mode=lax.GatherScatterMode.PROMISE_IN_BOUNDS)


_ROWS_PER_STEP = 8


def _normalize_one_row(rows_v, r, lane):
    v0 = rows_v[r, pl.ds(0, LANES)]
    v1 = rows_v[r, pl.ds(LANES, LANES)]
    v2 = rows_v[r, pl.ds(2 * LANES, LANES)]
    v3 = rows_v[r, pl.ds(3 * LANES, LANES)]
    p = v0 * v0 + v1 * v1 + v2 * v2 + v3 * v3
    p = p + _shuffle(p, lane ^ 8)
    p = p + _shuffle(p, lane ^ 4)
    p = p + _shuffle(p, lane ^ 2)
    ssv = p + _shuffle(p, lane ^ 1)
    y = plsc.bitcast(0x5F3759DF - (plsc.bitcast(ssv, jnp.int32) >> 1),
                     jnp.float32)
    h = ssv * 0.5
    y = y * (1.5 - h * y * y)
    y = y * (1.5 - h * y * y)
    y = y * (1.5 - h * y * y)
    y = jnp.minimum(y, 1e12)
    rows_v[r, pl.ds(0, LANES)] = v0 * y
    rows_v[r, pl.ds(LANES, LANES)] = v1 * y
    rows_v[r, pl.ds(2 * LANES, LANES)] = v2 * y
    rows_v[r, pl.ds(3 * LANES, LANES)] = v3 * y


def _normalize_rows(rows_v, n_rows):
    lane = lax.iota(jnp.int32, LANES)

    @plsc.parallel_loop(0, n_rows, _ROWS_PER_STEP, unroll=2)
    def body(r):
        for j in range(_ROWS_PER_STEP):
            _normalize_one_row(rows_v, r + j, lane)


def _two_tower_sc(c_ids, p_ids, c_tab, p_tab, u_out, i_out,
                  cidx_v, pidx_v, crows_v, prows_v, csem, psem):
    info = plsc.get_sparse_core_info()
    nc = info.num_cores
    wid = lax.axis_index("s") * nc + lax.axis_index("c")
    b_per_w = BATCH // (nc * info.num_subcores)
    n_chunks = b_per_w // CHUNK
    base = wid * b_per_w

    for c in range(n_chunks):
        pltpu.sync_copy(c_ids.at[pl.ds(base + c * CHUNK, CHUNK)],
                        cidx_v.at[c])
        pltpu.sync_copy(p_ids.at[pl.ds(base + c * CHUNK, CHUNK)],
                        pidx_v.at[c])

    work = []
    for c in range(n_chunks):
        work.append((c_tab, cidx_v, crows_v, csem, u_out, c))
        work.append((p_tab, pidx_v, prows_v, psem, i_out, c))

    def fire(item):
        tab, idx_v, rows_v, sem, _, c = item
        return pltpu.async_copy(tab.at[idx_v.at[c]], rows_v.at[c % 2], sem)

    copies = [fire(work[0]), fire(work[1])]
    for k, item in enumerate(work):
        _, _, rows_v, _, out, c = item
        copies[0].wait()
        copies.pop(0)
        if k + 2 < len(work):
            copies.append(fire(work[k + 2]))
        buf = rows_v.at[c % 2]
        _normalize_rows(buf, CHUNK)
        pltpu.sync_copy(buf.at[:, pl.ds(0, EMBED_DIM)],
                        out.at[pl.ds(base + c * CHUNK, CHUNK)])


def kernel(claimant_ids, provider_ids, claimant_table, provider_table):
    info = plsc.get_sparse_core_info()
    b_per_w = BATCH // (info.num_cores * info.num_subcores)
    n_chunks = b_per_w // CHUNK
    mesh = plsc.VectorSubcoreMesh(core_axis_name="c", subcore_axis_name="s")
    out_type = (
        jax.ShapeDtypeStruct((BATCH, EMBED_DIM), jnp.float32),
        jax.ShapeDtypeStruct((BATCH, EMBED_DIM), jnp.float32),
    )
    run = pl.kernel(
        _two_tower_sc,
        mesh=mesh,
        out_type=out_type,
        scratch_types=[
            pltpu.VMEM((n_chunks, CHUNK), jnp.int32),
            pltpu.VMEM((n_chunks, CHUNK), jnp.int32),
            pltpu.VMEM((2, CHUNK, PAD_DIM), jnp.float32),
            pltpu.VMEM((2, CHUNK, PAD_DIM), jnp.float32),
            pltpu.SemaphoreType.DMA,
            pltpu.SemaphoreType.DMA,
        ],
        compiler_params=pltpu.CompilerParams(
            needs_layout_passes=False, use_tc_tiling_on_sc=False),
    )
    c_tab = jnp.pad(claimant_table, ((0, 0), (0, PAD_DIM - EMBED_DIM)))
    p_tab = jnp.pad(provider_table, ((0, 0), (0, PAD_DIM - EMBED_DIM)))
    return run(claimant_ids.astype(jnp.int32), provider_ids.astype(jnp.int32),
               c_tab, p_tab)
